# baseline (speedup 1.0000x reference)
import functools

import jax
import jax.numpy as jnp
from jax import lax
from jax.experimental import pallas as pl
from jax.experimental.pallas import tpu as pltpu
from jax.experimental.pallas import tpu_sc as plsc

D = 64
SCALE = 8.0
NC = 2
NS = 16
NW = NC * NS
VOCAB = 1000000
B_TOTAL = 4096 * 200
B_PER_W = B_TOTAL // NW
GATHER = 128
KSUB = 2
CHUNK = GATHER * KSUB
NCHUNK = B_PER_W // CHUNK
IDX_ROWS_PER_W = B_PER_W // GATHER

_mesh = plsc.VectorSubcoreMesh(core_axis_name="c", subcore_axis_name="s")


@functools.partial(
    pl.kernel,
    mesh=_mesh,
    out_type=jax.ShapeDtypeStruct((B_TOTAL // 2, 2 * D), jnp.float32),
    scratch_types=[
        pltpu.VMEM((2, KSUB, GATHER), jnp.int32),
        pltpu.VMEM((2, CHUNK), jnp.int32),
        pltpu.VMEM((2, CHUNK, 2 * D), jnp.float32),
        pltpu.VMEM((2, CHUNK // 2, 2 * D), jnp.float32),
        pltpu.SemaphoreType.DMA,
        pltpu.SemaphoreType.DMA,
    ],
)
def _emb_lookup(qidx_hbm, half_hbm, table_hbm, out_hbm,
                idx_v, half_v, rows_v, out_v, g0, g1):
    wid = lax.axis_index("s") * NC + lax.axis_index("c")
    base = wid * B_PER_W
    idx_base = wid * IDX_ROWS_PER_W
    pbase = wid * (B_PER_W // 2)
    gsems = (g0, g1)

    def fire(i, b):
        pltpu.sync_copy(qidx_hbm.at[pl.ds(idx_base + i * KSUB, KSUB)],
                        idx_v.at[b])
        pltpu.sync_copy(half_hbm.at[pl.ds(base + i * CHUNK, CHUNK)],
                        half_v.at[b])
        for j in range(KSUB):
            pltpu.async_copy(table_hbm.at[idx_v.at[b, j]],
                             rows_v.at[b, pl.ds(j * GATHER, GATHER)],
                             gsems[b])

    def process(i, b):
        pltpu.make_async_copy(out_hbm.at[pl.ds(pbase, CHUNK)],
                              rows_v.at[b], gsems[b]).wait()

        def repack_group(g, carry):
            hv = half_v[b, pl.ds(g * 16, 16)]
            for u in range(8):
                p = g * 8 + u
                w0 = g * 16 + 2 * u
                w1 = w0 + 1
                j0 = hv[2 * u]
                j1 = hv[2 * u + 1]
                for c in range(D // 16):
                    out_v[b, p, pl.ds(c * 16, 16)] = (
                        rows_v[b, w0, pl.ds(j0 + c * 16, 16)] * SCALE)
                    out_v[b, p, pl.ds(D + c * 16, 16)] = (
                        rows_v[b, w1, pl.ds(j1 + c * 16, 16)] * SCALE)
            return carry

        lax.fori_loop(0, CHUNK // 16, repack_group, 0)
        pltpu.sync_copy(out_v.at[b],
                        out_hbm.at[pl.ds(pbase + i * (CHUNK // 2), CHUNK // 2)])

    fire(0, 0)

    def outer(t, carry):
        i0 = 2 * t
        fire(i0 + 1, 1)
        process(i0, 0)

        @pl.when(t + 1 < NCHUNK // 2)
        def _():
            fire(i0 + 2, 0)

        process(i0 + 1, 1)
        return carry

    lax.fori_loop(0, NCHUNK // 2, outer, 0)


def kernel(x, table):
    xf = x.astype(jnp.int32).reshape(-1)
    qidx = (xf >> 1).reshape(B_TOTAL // GATHER, GATHER)
    half = (xf & 1) * D
    table2 = table.reshape(VOCAB // 2, 2 * D)
    packed = _emb_lookup(qidx, half, table2)
    return packed.reshape(x.shape[0], x.shape[1], D)
